# baseline (speedup 1.0000x reference)
import jax
import jax.numpy as jnp
from jax import lax
from jax.experimental import pallas as pl
from jax.experimental.pallas import tpu as pltpu
from jax.experimental.pallas import tpu_sc as plsc

N = 10000
E = 320000
D_IN = 128
D_H = 32
D_OUT = 128

NUM_CORES = 2
NUM_SUBCORES = 16
NTILES = NUM_CORES * NUM_SUBCORES
CHUNK = 128
NCK = E // CHUNK
CPT = NCK // NTILES
XTRA = NCK - NTILES * CPT
CPT1 = CPT + 1
NPAIR = CPT // 2
TBL = 10240
RPT = TBL // NUM_SUBCORES
DEG_W = 8
ROWS_TC = 2000
GRID = N // ROWS_TC

_mesh = plsc.VectorSubcoreMesh(core_axis_name="c", subcore_axis_name="s")
_sc_params = pltpu.CompilerParams(use_tc_tiling_on_sc=False)


def _deg_body(edges_hbm, ones_hbm, zeros_hbm, out_hbm, dst_v, ones_v, z_v,
              table, ssem):
    c = lax.axis_index("c")
    s = lax.axis_index("s")
    wid = c * NUM_SUBCORES + s
    r0 = s * RPT
    base = wid * CPT + jnp.minimum(wid, XTRA)
    cnt = jnp.where(wid < XTRA, CPT1, CPT)
    pltpu.sync_copy(ones_hbm, ones_v)
    pltpu.sync_copy(zeros_hbm, z_v)
    for k in range(RPT // CHUNK):
        pltpu.sync_copy(z_v, table.at[pl.ds(r0 + k * CHUNK, CHUNK)])
    pltpu.sync_copy(edges_hbm.at[1, pl.ds(base, CPT)], dst_v.at[pl.ds(0, CPT)])

    @pl.when(wid < XTRA)
    def _():
        pltpu.sync_copy(edges_hbm.at[1, pl.ds(base + CPT, 1)],
                        dst_v.at[pl.ds(CPT, 1)])

    plsc.subcore_barrier()

    def body(j, carry):
        pltpu.async_copy(ones_v, table.at[dst_v.at[j]], ssem, add=True)
        return carry

    lax.fori_loop(0, cnt, body, 0)

    def wbody(j, carry):
        pltpu.make_async_copy(ones_v, table.at[dst_v.at[0]], ssem).wait()
        return carry

    lax.fori_loop(0, cnt, wbody, 0)
    plsc.subcore_barrier()
    pltpu.sync_copy(table.at[pl.ds(r0, RPT)], out_hbm.at[c, pl.ds(r0, RPT)])


_deg_call = pl.kernel(
    _deg_body,
    mesh=_mesh,
    compiler_params=_sc_params,
    out_type=jax.ShapeDtypeStruct((NUM_CORES, TBL, DEG_W), jnp.float32),
    scratch_types=[
        pltpu.VMEM((CPT1, CHUNK), jnp.int32),
        pltpu.VMEM((CHUNK, DEG_W), jnp.float32),
        pltpu.VMEM((CHUNK, DEG_W), jnp.float32),
        pltpu.VMEM_SHARED((TBL, DEG_W), jnp.float32),
        pltpu.SemaphoreType.DMA,
    ],
)


NBUF = 3


def _agg_body(tbl_hbm, edges_hbm, zeros_hbm, out_hbm,
              src_v, dst_v, rows_bufs, z_v, table, tblsh, gsems, ssems):
    c = lax.axis_index("c")
    s = lax.axis_index("s")
    wid = c * NUM_SUBCORES + s
    r0 = s * RPT
    base = wid * CPT + jnp.minimum(wid, XTRA)
    cnt = jnp.where(wid < XTRA, CPT1, CPT)
    pltpu.sync_copy(tbl_hbm.at[pl.ds(r0, RPT)], tblsh.at[pl.ds(r0, RPT)])
    pltpu.sync_copy(zeros_hbm, z_v)
    for k in range(RPT // CHUNK):
        pltpu.sync_copy(z_v, table.at[pl.ds(r0 + k * CHUNK, CHUNK)])
    pltpu.sync_copy(edges_hbm.at[0, pl.ds(base, CPT)], src_v.at[pl.ds(0, CPT)])
    pltpu.sync_copy(edges_hbm.at[1, pl.ds(base, CPT)], dst_v.at[pl.ds(0, CPT)])

    @pl.when(wid < XTRA)
    def _():
        pltpu.sync_copy(edges_hbm.at[0, pl.ds(base + CPT, 1)],
                        src_v.at[pl.ds(CPT, 1)])
        pltpu.sync_copy(edges_hbm.at[1, pl.ds(base + CPT, 1)],
                        dst_v.at[pl.ds(CPT, 1)])

    plsc.subcore_barrier()

    def gather(chunk, rows, sem):
        return pltpu.async_copy(tblsh.at[src_v.at[chunk]], rows, sem)

    def gather_wait(rows, sem):
        pltpu.make_async_copy(tblsh.at[src_v.at[0]], rows, sem).wait()

    def scatter(chunk, rows, sem):
        return pltpu.async_copy(rows, table.at[dst_v.at[chunk]], sem, add=True)

    for b in range(NBUF):
        gather(b, rows_bufs[b], gsems[b])
    last = cnt - 1

    def body(j, carry):
        c0 = NBUF * j
        handles = []
        for b in range(NBUF):
            gather_wait(rows_bufs[b], gsems[b])
            handles.append(scatter(c0 + b, rows_bufs[b], ssems[b]))
        for b in range(NBUF):
            handles[b].wait()
            gather(jnp.minimum(c0 + b + NBUF, last), rows_bufs[b], gsems[b])
        return carry

    lax.fori_loop(0, CPT // NBUF, body, 0)
    gather_wait(rows_bufs[0], gsems[0])

    @pl.when(wid < XTRA)
    def _():
        scatter(CPT, rows_bufs[0], ssems[0]).wait()

    for b in range(1, NBUF):
        gather_wait(rows_bufs[b], gsems[b])
    plsc.subcore_barrier()
    pltpu.sync_copy(table.at[pl.ds(r0, RPT)], out_hbm.at[c, pl.ds(r0, RPT)])


_agg_call = pl.kernel(
    _agg_body,
    mesh=_mesh,
    compiler_params=_sc_params,
    out_type=jax.ShapeDtypeStruct((NUM_CORES, TBL, D_H), jnp.float32),
    scratch_types=[
        pltpu.VMEM((CPT1, CHUNK), jnp.int32),
        pltpu.VMEM((CPT1, CHUNK), jnp.int32),
        [pltpu.VMEM((CHUNK, D_H), jnp.float32) for _ in range(NBUF)],
        pltpu.VMEM((CHUNK, D_H), jnp.float32),
        pltpu.VMEM_SHARED((TBL, D_H), jnp.float32),
        pltpu.VMEM_SHARED((TBL, D_H), jnp.float32),
        [pltpu.SemaphoreType.DMA for _ in range(NBUF)],
        [pltpu.SemaphoreType.DMA for _ in range(NBUF)],
    ],
)


def _k1_body(x_ref, w1_ref, degp_ref, hs_ref, dinv_ref):
    deg = degp_ref[0] + degp_ref[1] + 1.0
    dinv = lax.rsqrt(deg[:, 0:1])
    h1 = jnp.dot(x_ref[...], w1_ref[...], preferred_element_type=jnp.float32)
    hs_ref[...] = h1 * dinv
    dinv_ref[...] = dinv


def _k2_body(aggp_ref, hs_ref, dinv_ref, b1_ref, g_ref):
    p = aggp_ref[0] + aggp_ref[1] + hs_ref[...]
    dinv = dinv_ref[...]
    z = jnp.maximum(dinv * p + b1_ref[...], 0.0)
    g_ref[...] = z * dinv


def _k3_body(aggp_ref, g_ref, dinv_ref, w2_ref, b2_ref, out_ref):
    a = dinv_ref[...] * (aggp_ref[0] + aggp_ref[1] + g_ref[...])
    out_ref[...] = (
        jnp.dot(a, w2_ref[...], preferred_element_type=jnp.float32) + b2_ref[...]
    )


_k1_call = pl.pallas_call(
    _k1_body,
    grid=(GRID,),
    in_specs=[
        pl.BlockSpec((ROWS_TC, D_IN), lambda i: (i, 0)),
        pl.BlockSpec((D_IN, D_H), lambda i: (0, 0)),
        pl.BlockSpec((NUM_CORES, ROWS_TC, DEG_W), lambda i: (0, i, 0)),
    ],
    out_specs=[
        pl.BlockSpec((ROWS_TC, D_H), lambda i: (i, 0)),
        pl.BlockSpec((ROWS_TC, 1), lambda i: (i, 0)),
    ],
    out_shape=[
        jax.ShapeDtypeStruct((TBL, D_H), jnp.float32),
        jax.ShapeDtypeStruct((N, 1), jnp.float32),
    ],
)

_k2_call = pl.pallas_call(
    _k2_body,
    grid=(GRID,),
    in_specs=[
        pl.BlockSpec((NUM_CORES, ROWS_TC, D_H), lambda i: (0, i, 0)),
        pl.BlockSpec((ROWS_TC, D_H), lambda i: (i, 0)),
        pl.BlockSpec((ROWS_TC, 1), lambda i: (i, 0)),
        pl.BlockSpec((1, D_H), lambda i: (0, 0)),
    ],
    out_specs=pl.BlockSpec((ROWS_TC, D_H), lambda i: (i, 0)),
    out_shape=jax.ShapeDtypeStruct((TBL, D_H), jnp.float32),
)

_k3_call = pl.pallas_call(
    _k3_body,
    grid=(GRID,),
    in_specs=[
        pl.BlockSpec((NUM_CORES, ROWS_TC, D_H), lambda i: (0, i, 0)),
        pl.BlockSpec((ROWS_TC, D_H), lambda i: (i, 0)),
        pl.BlockSpec((ROWS_TC, 1), lambda i: (i, 0)),
        pl.BlockSpec((D_H, D_OUT), lambda i: (0, 0)),
        pl.BlockSpec((1, D_OUT), lambda i: (0, 0)),
    ],
    out_specs=pl.BlockSpec((ROWS_TC, D_OUT), lambda i: (i, 0)),
    out_shape=jax.ShapeDtypeStruct((N, D_OUT), jnp.float32),
)


def kernel(x, edge_index, W1, b1, W2, b2):
    edges = edge_index.astype(jnp.int32).reshape(2, NCK, CHUNK)
    ones8 = jnp.ones((CHUNK, DEG_W), jnp.float32)
    zeros8 = jnp.zeros((CHUNK, DEG_W), jnp.float32)
    zeros32 = jnp.zeros((CHUNK, D_H), jnp.float32)

    degp = _deg_call(edges, ones8, zeros8)
    hs, dinv = _k1_call(x, W1, degp)
    agg1p = _agg_call(hs, edges, zeros32)
    g = _k2_call(agg1p, hs, dinv, b1.reshape(1, D_H))
    agg2p = _agg_call(g, edges, zeros32)
    return _k3_call(agg2p, g, dinv, W2, b2.reshape(1, D_OUT))


# speedup vs baseline: 1.0081x; 1.0081x over previous
import jax
import jax.numpy as jnp
from jax import lax
from jax.experimental import pallas as pl
from jax.experimental.pallas import tpu as pltpu
from jax.experimental.pallas import tpu_sc as plsc

N = 10000
E = 320000
D_IN = 128
D_H = 32
D_OUT = 128

NUM_CORES = 2
NUM_SUBCORES = 16
NTILES = NUM_CORES * NUM_SUBCORES
CHUNK = 128
NCK = E // CHUNK
CPT = NCK // NTILES
XTRA = NCK - NTILES * CPT
CPT1 = CPT + 1
NPAIR = CPT // 2
TBL = 10240
RPT = TBL // NUM_SUBCORES
DEG_W = 8
ROWS_TC = 5000
GRID = N // ROWS_TC

_mesh = plsc.VectorSubcoreMesh(core_axis_name="c", subcore_axis_name="s")
_sc_params = pltpu.CompilerParams(use_tc_tiling_on_sc=False)


def _deg_body(edges_hbm, ones_hbm, zeros_hbm, out_hbm, dst_v, ones_v, z_v,
              table, ssem):
    c = lax.axis_index("c")
    s = lax.axis_index("s")
    wid = c * NUM_SUBCORES + s
    r0 = s * RPT
    base = wid * CPT + jnp.minimum(wid, XTRA)
    cnt = jnp.where(wid < XTRA, CPT1, CPT)
    pltpu.sync_copy(ones_hbm, ones_v)
    pltpu.sync_copy(zeros_hbm, z_v)
    for k in range(RPT // CHUNK):
        pltpu.sync_copy(z_v, table.at[pl.ds(r0 + k * CHUNK, CHUNK)])
    pltpu.sync_copy(edges_hbm.at[1, pl.ds(base, CPT)], dst_v.at[pl.ds(0, CPT)])

    @pl.when(wid < XTRA)
    def _():
        pltpu.sync_copy(edges_hbm.at[1, pl.ds(base + CPT, 1)],
                        dst_v.at[pl.ds(CPT, 1)])

    plsc.subcore_barrier()

    def body(j, carry):
        pltpu.async_copy(ones_v, table.at[dst_v.at[j]], ssem, add=True)
        return carry

    lax.fori_loop(0, cnt, body, 0)

    def wbody(j, carry):
        pltpu.make_async_copy(ones_v, table.at[dst_v.at[0]], ssem).wait()
        return carry

    lax.fori_loop(0, cnt, wbody, 0)
    plsc.subcore_barrier()
    pltpu.sync_copy(table.at[pl.ds(r0, RPT)], out_hbm.at[c, pl.ds(r0, RPT)])


_deg_call = pl.kernel(
    _deg_body,
    mesh=_mesh,
    compiler_params=_sc_params,
    out_type=jax.ShapeDtypeStruct((NUM_CORES, TBL, DEG_W), jnp.float32),
    scratch_types=[
        pltpu.VMEM((CPT1, CHUNK), jnp.int32),
        pltpu.VMEM((CHUNK, DEG_W), jnp.float32),
        pltpu.VMEM((CHUNK, DEG_W), jnp.float32),
        pltpu.VMEM_SHARED((TBL, DEG_W), jnp.float32),
        pltpu.SemaphoreType.DMA,
    ],
)


NBUF = 6


def _agg_body(tbl_hbm, edges_hbm, zeros_hbm, out_hbm,
              src_v, dst_v, rows_bufs, z_v, table, tblsh, gsems, ssems):
    c = lax.axis_index("c")
    s = lax.axis_index("s")
    wid = c * NUM_SUBCORES + s
    r0 = s * RPT
    base = wid * CPT + jnp.minimum(wid, XTRA)
    cnt = jnp.where(wid < XTRA, CPT1, CPT)
    pltpu.sync_copy(tbl_hbm.at[pl.ds(r0, RPT)], tblsh.at[pl.ds(r0, RPT)])
    pltpu.sync_copy(zeros_hbm, z_v)
    for k in range(RPT // CHUNK):
        pltpu.sync_copy(z_v, table.at[pl.ds(r0 + k * CHUNK, CHUNK)])
    pltpu.sync_copy(edges_hbm.at[0, pl.ds(base, CPT)], src_v.at[pl.ds(0, CPT)])
    pltpu.sync_copy(edges_hbm.at[1, pl.ds(base, CPT)], dst_v.at[pl.ds(0, CPT)])

    @pl.when(wid < XTRA)
    def _():
        pltpu.sync_copy(edges_hbm.at[0, pl.ds(base + CPT, 1)],
                        src_v.at[pl.ds(CPT, 1)])
        pltpu.sync_copy(edges_hbm.at[1, pl.ds(base + CPT, 1)],
                        dst_v.at[pl.ds(CPT, 1)])

    plsc.subcore_barrier()

    def gather(chunk, rows, sem):
        return pltpu.async_copy(tblsh.at[src_v.at[chunk]], rows, sem)

    def gather_wait(rows, sem):
        pltpu.make_async_copy(tblsh.at[src_v.at[0]], rows, sem).wait()

    def scatter(chunk, rows, sem):
        return pltpu.async_copy(rows, table.at[dst_v.at[chunk]], sem, add=True)

    for b in range(NBUF):
        gather(b, rows_bufs[b], gsems[b])
    last = cnt - 1

    def body(j, carry):
        c0 = NBUF * j
        handles = []
        for b in range(NBUF):
            gather_wait(rows_bufs[b], gsems[b])
            handles.append(scatter(c0 + b, rows_bufs[b], ssems[b]))
        for b in range(NBUF):
            handles[b].wait()
            gather(jnp.minimum(c0 + b + NBUF, last), rows_bufs[b], gsems[b])
        return carry

    lax.fori_loop(0, CPT // NBUF, body, 0)
    gather_wait(rows_bufs[0], gsems[0])

    @pl.when(wid < XTRA)
    def _():
        scatter(CPT, rows_bufs[0], ssems[0]).wait()

    for b in range(1, NBUF):
        gather_wait(rows_bufs[b], gsems[b])
    plsc.subcore_barrier()
    pltpu.sync_copy(table.at[pl.ds(r0, RPT)], out_hbm.at[c, pl.ds(r0, RPT)])


_agg_call = pl.kernel(
    _agg_body,
    mesh=_mesh,
    compiler_params=_sc_params,
    out_type=jax.ShapeDtypeStruct((NUM_CORES, TBL, D_H), jnp.float32),
    scratch_types=[
        pltpu.VMEM((CPT1, CHUNK), jnp.int32),
        pltpu.VMEM((CPT1, CHUNK), jnp.int32),
        [pltpu.VMEM((CHUNK, D_H), jnp.float32) for _ in range(NBUF)],
        pltpu.VMEM((CHUNK, D_H), jnp.float32),
        pltpu.VMEM_SHARED((TBL, D_H), jnp.float32),
        pltpu.VMEM_SHARED((TBL, D_H), jnp.float32),
        [pltpu.SemaphoreType.DMA for _ in range(NBUF)],
        [pltpu.SemaphoreType.DMA for _ in range(NBUF)],
    ],
)


def _k1_body(x_ref, w1_ref, degp_ref, hs_ref, dinv_ref):
    deg = degp_ref[0] + degp_ref[1] + 1.0
    dinv = lax.rsqrt(deg[:, 0:1])
    h1 = jnp.dot(x_ref[...], w1_ref[...], preferred_element_type=jnp.float32)
    hs_ref[...] = h1 * dinv
    dinv_ref[...] = dinv


def _k2_body(aggp_ref, hs_ref, dinv_ref, b1_ref, g_ref):
    p = aggp_ref[0] + aggp_ref[1] + hs_ref[...]
    dinv = dinv_ref[...]
    z = jnp.maximum(dinv * p + b1_ref[...], 0.0)
    g_ref[...] = z * dinv


def _k3_body(aggp_ref, g_ref, dinv_ref, w2_ref, b2_ref, out_ref):
    a = dinv_ref[...] * (aggp_ref[0] + aggp_ref[1] + g_ref[...])
    out_ref[...] = (
        jnp.dot(a, w2_ref[...], preferred_element_type=jnp.float32) + b2_ref[...]
    )


_k1_call = pl.pallas_call(
    _k1_body,
    grid=(GRID,),
    in_specs=[
        pl.BlockSpec((ROWS_TC, D_IN), lambda i: (i, 0)),
        pl.BlockSpec((D_IN, D_H), lambda i: (0, 0)),
        pl.BlockSpec((NUM_CORES, ROWS_TC, DEG_W), lambda i: (0, i, 0)),
    ],
    out_specs=[
        pl.BlockSpec((ROWS_TC, D_H), lambda i: (i, 0)),
        pl.BlockSpec((ROWS_TC, 1), lambda i: (i, 0)),
    ],
    out_shape=[
        jax.ShapeDtypeStruct((TBL, D_H), jnp.float32),
        jax.ShapeDtypeStruct((N, 1), jnp.float32),
    ],
)

_k2_call = pl.pallas_call(
    _k2_body,
    grid=(GRID,),
    in_specs=[
        pl.BlockSpec((NUM_CORES, ROWS_TC, D_H), lambda i: (0, i, 0)),
        pl.BlockSpec((ROWS_TC, D_H), lambda i: (i, 0)),
        pl.BlockSpec((ROWS_TC, 1), lambda i: (i, 0)),
        pl.BlockSpec((1, D_H), lambda i: (0, 0)),
    ],
    out_specs=pl.BlockSpec((ROWS_TC, D_H), lambda i: (i, 0)),
    out_shape=jax.ShapeDtypeStruct((TBL, D_H), jnp.float32),
)

_k3_call = pl.pallas_call(
    _k3_body,
    grid=(GRID,),
    in_specs=[
        pl.BlockSpec((NUM_CORES, ROWS_TC, D_H), lambda i: (0, i, 0)),
        pl.BlockSpec((ROWS_TC, D_H), lambda i: (i, 0)),
        pl.BlockSpec((ROWS_TC, 1), lambda i: (i, 0)),
        pl.BlockSpec((D_H, D_OUT), lambda i: (0, 0)),
        pl.BlockSpec((1, D_OUT), lambda i: (0, 0)),
    ],
    out_specs=pl.BlockSpec((ROWS_TC, D_OUT), lambda i: (i, 0)),
    out_shape=jax.ShapeDtypeStruct((N, D_OUT), jnp.float32),
)


def kernel(x, edge_index, W1, b1, W2, b2):
    edges = edge_index.astype(jnp.int32).reshape(2, NCK, CHUNK)
    ones8 = jnp.ones((CHUNK, DEG_W), jnp.float32)
    zeros8 = jnp.zeros((CHUNK, DEG_W), jnp.float32)
    zeros32 = jnp.zeros((CHUNK, D_H), jnp.float32)

    degp = _deg_call(edges, ones8, zeros8)
    hs, dinv = _k1_call(x, W1, degp)
    agg1p = _agg_call(hs, edges, zeros32)
    g = _k2_call(agg1p, hs, dinv, b1.reshape(1, D_H))
    agg2p = _agg_call(g, edges, zeros32)
    return _k3_call(agg2p, g, dinv, W2, b2.reshape(1, D_OUT))
